# Initial kernel scaffold; baseline (speedup 1.0000x reference)
import jax
import jax.numpy as jnp
from jax.experimental import pallas as pl


def kernel(x, edge_index, W1l, W1r, b1, g1, be1, W2l, W2r, b2, g2, be2, W3l, W3r, b3):
    raise NotImplementedError("write your pallas kernel here")


# speedup vs baseline: 6.2963x; 6.2963x over previous
import functools

import jax
import jax.numpy as jnp
from jax import lax
from jax.experimental import pallas as pl
from jax.experimental.pallas import tpu as pltpu
from jax.experimental.pallas import tpu_sc as plsc

NC = 2
NS = 16
NW = NC * NS
CH = 128


def _seg_kernel(n_pad, e_pad, d, with_deg):
    zrows = n_pad // NS
    wrows = e_pad // (NW * CH)
    mesh = plsc.VectorSubcoreMesh(core_axis_name="c", subcore_axis_name="s")

    out_type = [jax.ShapeDtypeStruct((NC, n_pad, d), jnp.float32)]
    scratch = [
        pltpu.VMEM_SHARED((n_pad, d), jnp.float32),
        pltpu.VMEM((CH,), jnp.int32),
        pltpu.VMEM((CH,), jnp.int32),
        pltpu.VMEM((CH, d), jnp.float32),
        pltpu.VMEM((zrows, d), jnp.float32),
    ]
    if with_deg:
        out_type.append(jax.ShapeDtypeStruct((NC * n_pad,), jnp.float32))
        scratch.append(pltpu.VMEM_SHARED((n_pad,), jnp.float32))
        scratch.append(pltpu.VMEM((CH,), jnp.float32))
        scratch.append(pltpu.VMEM((zrows,), jnp.float32))

    def body(y_h, src_h, dst_h, z2_h, z1_h, part_h, *rest):
        if with_deg:
            deg_h, acc, si, di, rows, stg, dacc, ones, dstg = rest
        else:
            acc, si, di, rows, stg = rest
        c = lax.axis_index("c")
        s = lax.axis_index("s")
        wid = s * NC + c
        r0 = s * zrows
        pltpu.sync_copy(z2_h.at[pl.ds(r0, zrows)], stg)
        pltpu.sync_copy(stg, acc.at[pl.ds(r0, zrows)])
        if with_deg:
            pltpu.sync_copy(z1_h.at[pl.ds(r0, zrows)], dstg)
            pltpu.sync_copy(dstg, dacc.at[pl.ds(r0, zrows)])
            for i in range(CH // 16):
                ones[pl.ds(i * 16, 16)] = jnp.ones((16,), jnp.float32)
        plsc.subcore_barrier()

        base = wid * wrows

        def step(j, carry):
            e0 = (base + j) * CH
            pltpu.sync_copy(src_h.at[pl.ds(e0, CH)], si)
            pltpu.sync_copy(dst_h.at[pl.ds(e0, CH)], di)
            pltpu.sync_copy(y_h.at[si], rows)
            pltpu.sync_copy(rows, acc.at[di], add=True)
            if with_deg:
                pltpu.sync_copy(ones, dacc.at[di], add=True)
            return carry

        lax.fori_loop(0, wrows, step, 0)
        plsc.subcore_barrier()
        pltpu.sync_copy(acc.at[pl.ds(r0, zrows)], stg)
        pltpu.sync_copy(stg, part_h.at[c, pl.ds(r0, zrows)])
        if with_deg:
            pltpu.sync_copy(dacc.at[pl.ds(r0, zrows)], dstg)
            pltpu.sync_copy(dstg, deg_h.at[pl.ds(c * n_pad + r0, zrows)])

    return pl.kernel(body, out_type=tuple(out_type), mesh=mesh,
                     scratch_types=scratch,
                     compiler_params=pltpu.CompilerParams(
                         use_tc_tiling_on_sc=False))


def _seg1_kernel(n_pad, e_pad):
    zrows = n_pad // NS
    wrows = e_pad // (NW * CH)
    mesh = plsc.VectorSubcoreMesh(core_axis_name="c", subcore_axis_name="s")

    def body(y_h, src_h, dst_h, z1_h, part_h, acc, si, di, vals, stg):
        c = lax.axis_index("c")
        s = lax.axis_index("s")
        wid = s * NC + c
        r0 = s * zrows
        pltpu.sync_copy(z1_h.at[pl.ds(r0, zrows)], stg)
        pltpu.sync_copy(stg, acc.at[pl.ds(r0, zrows)])
        plsc.subcore_barrier()
        base = wid * wrows

        def step(j, carry):
            e0 = (base + j) * CH
            pltpu.sync_copy(src_h.at[pl.ds(e0, CH)], si)
            pltpu.sync_copy(dst_h.at[pl.ds(e0, CH)], di)
            pltpu.sync_copy(y_h.at[si], vals)
            pltpu.sync_copy(vals, acc.at[di], add=True)
            return carry

        lax.fori_loop(0, wrows, step, 0)
        plsc.subcore_barrier()
        pltpu.sync_copy(acc.at[pl.ds(r0, zrows)], stg)
        pltpu.sync_copy(stg, part_h.at[pl.ds(c * n_pad + r0, zrows)])

    return pl.kernel(
        body,
        out_type=jax.ShapeDtypeStruct((NC * n_pad,), jnp.float32),
        mesh=mesh,
        scratch_types=[
            pltpu.VMEM_SHARED((n_pad,), jnp.float32),
            pltpu.VMEM((CH,), jnp.int32),
            pltpu.VMEM((CH,), jnp.int32),
            pltpu.VMEM((CH,), jnp.float32),
            pltpu.VMEM((zrows,), jnp.float32),
        ],
        compiler_params=pltpu.CompilerParams(use_tc_tiling_on_sc=False))


def _k1_body(x_ref, wl_ref, wr_ref, b_ref, y_ref, r_ref):
    x = x_ref[...]
    y_ref[...] = jnp.dot(x, wl_ref[...], preferred_element_type=jnp.float32)
    r_ref[...] = (jnp.dot(x, wr_ref[...], preferred_element_type=jnp.float32)
                  + b_ref[...][None, :])


def _k2_body(n, p_ref, d_ref, r1_ref, g_ref, be_ref, wl_ref, wr_ref, b2_ref,
             x1_ref, y2_ref, r2_ref, inv_ref):
    agg = p_ref[0, :n, :] + p_ref[1, :n, :]
    deg = d_ref[0, :n] + d_ref[1, :n]
    inv = 1.0 / jnp.maximum(deg, 1.0)
    t = agg * inv[:, None] + r1_ref[...]
    mu = jnp.mean(t, axis=0)
    var = jnp.mean((t - mu[None, :]) ** 2, axis=0)
    xh = (t - mu[None, :]) * lax.rsqrt(var + 1e-5)[None, :]
    x1 = jnp.maximum(xh * g_ref[...][None, :] + be_ref[...][None, :], 0.0)
    x1_ref[...] = x1
    y2_ref[...] = jnp.dot(x1, wl_ref[...], preferred_element_type=jnp.float32)
    r2_ref[...] = (jnp.dot(x1, wr_ref[...], preferred_element_type=jnp.float32)
                   + b2_ref[...][None, :])
    inv_ref[...] = inv


def _k3_body(n, p_ref, inv_ref, r2_ref, x1_ref, g_ref, be_ref, w3l_ref,
             w3r_ref, b3_ref, y3_ref, r3_ref):
    agg = p_ref[0, :n, :] + p_ref[1, :n, :]
    inv = inv_ref[...]
    t = agg * inv[:, None] + r2_ref[...]
    mu = jnp.mean(t, axis=0)
    var = jnp.mean((t - mu[None, :]) ** 2, axis=0)
    xh = (t - mu[None, :]) * lax.rsqrt(var + 1e-5)[None, :]
    x2 = jnp.maximum(xh * g_ref[...][None, :] + be_ref[...][None, :], 0.0)
    x2 = x2 + x1_ref[...]
    y3_ref[...] = jnp.sum(x2 * w3l_ref[0][None, :], axis=1)
    r3_ref[...] = jnp.sum(x2 * w3r_ref[0][None, :], axis=1) + b3_ref[0]


def _k4_body(n, p_ref, inv_ref, r3_ref, o_ref):
    agg = p_ref[0, :n] + p_ref[1, :n]
    o_ref[...] = agg * inv_ref[...] + r3_ref[...]


def kernel(x, edge_index, W1l, W1r, b1, g1, be1, W2l, W2r, b2, g2, be2,
           W3l, W3r, b3):
    n, d_in = x.shape
    d_h = W1l.shape[0]
    e = edge_index.shape[1]

    epw = -(-e // (NW * CH)) * CH
    e_pad = epw * NW
    n_pad = -(-(n + 1) // (NS * 8)) * (NS * 8)
    src = jnp.concatenate(
        [edge_index[0], jnp.zeros((e_pad - e,), jnp.int32)])
    dst = jnp.concatenate(
        [edge_index[1], jnp.full((e_pad - e,), n, jnp.int32)])
    z2 = jnp.zeros((n_pad, d_h), jnp.float32)
    z1 = jnp.zeros((n_pad,), jnp.float32)

    seg_d = _seg_kernel(n_pad, e_pad, d_h, True)
    seg = _seg_kernel(n_pad, e_pad, d_h, False)
    seg1 = _seg1_kernel(n_pad, e_pad)

    f32 = jnp.float32
    k1 = pl.pallas_call(
        _k1_body,
        out_shape=(jax.ShapeDtypeStruct((n, d_h), f32),
                   jax.ShapeDtypeStruct((n, d_h), f32)))
    y1, r1 = k1(x, W1l.T, W1r.T, b1)

    part1, degp = seg_d(y1, src, dst, z2, z1)
    degp = degp.reshape(NC, n_pad)

    k2 = pl.pallas_call(
        functools.partial(_k2_body, n),
        out_shape=(jax.ShapeDtypeStruct((n, d_h), f32),
                   jax.ShapeDtypeStruct((n, d_h), f32),
                   jax.ShapeDtypeStruct((n, d_h), f32),
                   jax.ShapeDtypeStruct((n,), f32)))
    x1, y2, r2, inv = k2(part1, degp, r1, g1, be1, W2l.T, W2r.T, b2)

    part2 = seg(y2, src, dst, z2, z1)[0]

    k3 = pl.pallas_call(
        functools.partial(_k3_body, n),
        out_shape=(jax.ShapeDtypeStruct((n,), f32),
                   jax.ShapeDtypeStruct((n,), f32)))
    y3, r3 = k3(part2, inv, r2, x1, g2, be2, W3l, W3r, b3)

    part3 = seg1(y3, src, dst, z1).reshape(NC, n_pad)

    k4 = pl.pallas_call(
        functools.partial(_k4_body, n),
        out_shape=jax.ShapeDtypeStruct((n,), f32))
    return k4(part3, inv, r3)


# speedup vs baseline: 8.0656x; 1.2810x over previous
import functools

import jax
import jax.numpy as jnp
from jax import lax
from jax.experimental import pallas as pl
from jax.experimental.pallas import tpu as pltpu
from jax.experimental.pallas import tpu_sc as plsc

NC = 2
NS = 16
NW = NC * NS
CH = 128
NBUF = 4

_SC_PARAMS = pltpu.CompilerParams(use_tc_tiling_on_sc=False)


def _seg_kernel(n_pad, e_pad, d, with_deg):
    zrows = n_pad // NS
    wrows = e_pad // (NW * CH)
    ngroups = wrows // NBUF
    mesh = plsc.VectorSubcoreMesh(core_axis_name="c", subcore_axis_name="s")

    zchunks = []
    off = 0
    while off < zrows:
        zchunks.append((off, min(CH, zrows - off)))
        off += CH

    out_type = [jax.ShapeDtypeStruct((NC, n_pad, d), jnp.float32)]
    scratch = [
        pltpu.VMEM_SHARED((n_pad, d), jnp.float32),
        pltpu.VMEM((wrows, CH), jnp.int32),
        pltpu.VMEM((wrows, CH), jnp.int32),
    ]
    scratch += [pltpu.VMEM((CH, d), jnp.float32) for _ in range(NBUF)]
    scratch += [pltpu.SemaphoreType.DMA for _ in range(NBUF)]
    if with_deg:
        out_type.append(jax.ShapeDtypeStruct((NC * n_pad,), jnp.float32))
        scratch.append(pltpu.VMEM_SHARED((n_pad,), jnp.float32))
        scratch.append(pltpu.VMEM((CH,), jnp.float32))
        scratch.append(pltpu.VMEM((zrows,), jnp.float32))

    def body(y_h, src_h, dst_h, z2_h, z1_h, part_h, *rest):
        if with_deg:
            deg_h = rest[0]
            rest = rest[1:]
        acc, si2, di2 = rest[:3]
        rows = rest[3:3 + NBUF]
        gsem = rest[3 + NBUF:3 + 2 * NBUF]
        if with_deg:
            dacc, ones, dstg = rest[3 + 2 * NBUF:]
        c = lax.axis_index("c")
        s = lax.axis_index("s")
        wid = s * NC + c
        r0 = s * zrows
        for zo, zs in zchunks:
            pltpu.sync_copy(z2_h.at[pl.ds(r0 + zo, zs)],
                            rows[0].at[pl.ds(0, zs)])
            pltpu.sync_copy(rows[0].at[pl.ds(0, zs)],
                            acc.at[pl.ds(r0 + zo, zs)])
        row0 = wid * wrows
        pltpu.sync_copy(src_h.at[pl.ds(row0, wrows)], si2)
        pltpu.sync_copy(dst_h.at[pl.ds(row0, wrows)], di2)
        if with_deg:
            pltpu.sync_copy(z1_h.at[pl.ds(r0, zrows)], dstg)
            pltpu.sync_copy(dstg, dacc.at[pl.ds(r0, zrows)])
            for i in range(CH // 16):
                ones[pl.ds(i * 16, 16)] = jnp.ones((16,), jnp.float32)
        plsc.subcore_barrier()

        for b in range(NBUF):
            pltpu.async_copy(y_h.at[si2.at[b]], rows[b], gsem[b])

        def group(g, carry):
            for b in range(NBUF):
                j = g * NBUF + b
                pltpu.make_async_copy(y_h.at[si2.at[b]], rows[b],
                                      gsem[b]).wait()
                pltpu.sync_copy(rows[b], acc.at[di2.at[j]], add=True)
                if with_deg:
                    pltpu.sync_copy(ones, dacc.at[di2.at[j]], add=True)

                @pl.when(g < ngroups - 1)
                def _():
                    pltpu.async_copy(y_h.at[si2.at[j + NBUF]], rows[b],
                                     gsem[b])
            return carry

        lax.fori_loop(0, ngroups, group, 0)
        plsc.subcore_barrier()
        for zo, zs in zchunks:
            pltpu.sync_copy(acc.at[pl.ds(r0 + zo, zs)],
                            rows[0].at[pl.ds(0, zs)])
            pltpu.sync_copy(rows[0].at[pl.ds(0, zs)],
                            part_h.at[c, pl.ds(r0 + zo, zs)])
        if with_deg:
            pltpu.sync_copy(dacc.at[pl.ds(r0, zrows)], dstg)
            pltpu.sync_copy(dstg, deg_h.at[pl.ds(c * n_pad + r0, zrows)])

    return pl.kernel(body, out_type=tuple(out_type), mesh=mesh,
                     scratch_types=scratch, compiler_params=_SC_PARAMS)


def _seg1_kernel(n_pad, e_pad):
    zrows = n_pad // NS
    wrows = e_pad // (NW * CH)
    ngroups = wrows // NBUF
    mesh = plsc.VectorSubcoreMesh(core_axis_name="c", subcore_axis_name="s")

    scratch = [
        pltpu.VMEM_SHARED((n_pad,), jnp.float32),
        pltpu.VMEM((wrows, CH), jnp.int32),
        pltpu.VMEM((wrows, CH), jnp.int32),
        pltpu.VMEM((zrows,), jnp.float32),
    ]
    scratch += [pltpu.VMEM((CH,), jnp.float32) for _ in range(NBUF)]
    scratch += [pltpu.SemaphoreType.DMA for _ in range(NBUF)]

    def body(y_h, src_h, dst_h, z1_h, part_h, *rest):
        acc, si2, di2, stg = rest[:4]
        vals = rest[4:4 + NBUF]
        gsem = rest[4 + NBUF:]
        c = lax.axis_index("c")
        s = lax.axis_index("s")
        wid = s * NC + c
        r0 = s * zrows
        pltpu.sync_copy(z1_h.at[pl.ds(r0, zrows)], stg)
        pltpu.sync_copy(stg, acc.at[pl.ds(r0, zrows)])
        row0 = wid * wrows
        pltpu.sync_copy(src_h.at[pl.ds(row0, wrows)], si2)
        pltpu.sync_copy(dst_h.at[pl.ds(row0, wrows)], di2)
        plsc.subcore_barrier()

        for b in range(NBUF):
            pltpu.async_copy(y_h.at[si2.at[b]], vals[b], gsem[b])

        def group(g, carry):
            for b in range(NBUF):
                j = g * NBUF + b
                pltpu.make_async_copy(y_h.at[si2.at[b]], vals[b],
                                      gsem[b]).wait()
                pltpu.sync_copy(vals[b], acc.at[di2.at[j]], add=True)

                @pl.when(g < ngroups - 1)
                def _():
                    pltpu.async_copy(y_h.at[si2.at[j + NBUF]], vals[b],
                                     gsem[b])
            return carry

        lax.fori_loop(0, ngroups, group, 0)
        plsc.subcore_barrier()
        pltpu.sync_copy(acc.at[pl.ds(r0, zrows)], stg)
        pltpu.sync_copy(stg, part_h.at[pl.ds(c * n_pad + r0, zrows)])

    return pl.kernel(
        body,
        out_type=jax.ShapeDtypeStruct((NC * n_pad,), jnp.float32),
        mesh=mesh, scratch_types=scratch, compiler_params=_SC_PARAMS)


def _k1_body(x_ref, wl_ref, wr_ref, b_ref, y_ref, r_ref):
    x = x_ref[...]
    y_ref[...] = jnp.dot(x, wl_ref[...], preferred_element_type=jnp.float32)
    r_ref[...] = (jnp.dot(x, wr_ref[...], preferred_element_type=jnp.float32)
                  + b_ref[...][None, :])


def _k2_body(n, p_ref, d_ref, r1_ref, g_ref, be_ref, wl_ref, wr_ref, b2_ref,
             x1_ref, y2_ref, r2_ref, inv_ref):
    agg = p_ref[0, :n, :] + p_ref[1, :n, :]
    deg = d_ref[0, :n] + d_ref[1, :n]
    inv = 1.0 / jnp.maximum(deg, 1.0)
    t = agg * inv[:, None] + r1_ref[...]
    mu = jnp.mean(t, axis=0)
    var = jnp.mean((t - mu[None, :]) ** 2, axis=0)
    xh = (t - mu[None, :]) * lax.rsqrt(var + 1e-5)[None, :]
    x1 = jnp.maximum(xh * g_ref[...][None, :] + be_ref[...][None, :], 0.0)
    x1_ref[...] = x1
    y2_ref[...] = jnp.dot(x1, wl_ref[...], preferred_element_type=jnp.float32)
    r2_ref[...] = (jnp.dot(x1, wr_ref[...], preferred_element_type=jnp.float32)
                   + b2_ref[...][None, :])
    inv_ref[...] = inv


def _k3_body(n, p_ref, inv_ref, r2_ref, x1_ref, g_ref, be_ref, w3l_ref,
             w3r_ref, b3_ref, y3_ref, r3_ref):
    agg = p_ref[0, :n, :] + p_ref[1, :n, :]
    inv = inv_ref[...]
    t = agg * inv[:, None] + r2_ref[...]
    mu = jnp.mean(t, axis=0)
    var = jnp.mean((t - mu[None, :]) ** 2, axis=0)
    xh = (t - mu[None, :]) * lax.rsqrt(var + 1e-5)[None, :]
    x2 = jnp.maximum(xh * g_ref[...][None, :] + be_ref[...][None, :], 0.0)
    x2 = x2 + x1_ref[...]
    y3_ref[...] = jnp.sum(x2 * w3l_ref[0][None, :], axis=1)
    r3_ref[...] = jnp.sum(x2 * w3r_ref[0][None, :], axis=1) + b3_ref[0]


def _k4_body(n, p_ref, inv_ref, r3_ref, o_ref):
    agg = p_ref[0, :n] + p_ref[1, :n]
    o_ref[...] = agg * inv_ref[...] + r3_ref[...]


def kernel(x, edge_index, W1l, W1r, b1, g1, be1, W2l, W2r, b2, g2, be2,
           W3l, W3r, b3):
    n, d_in = x.shape
    d_h = W1l.shape[0]
    e = edge_index.shape[1]

    epw = -(-e // (NW * CH * NBUF)) * (CH * NBUF)
    e_pad = epw * NW
    n_pad = -(-(n + 1) // (NS * 8)) * (NS * 8)
    src = jnp.concatenate(
        [edge_index[0], jnp.zeros((e_pad - e,), jnp.int32)]).reshape(-1, CH)
    dst = jnp.concatenate(
        [edge_index[1], jnp.full((e_pad - e,), n, jnp.int32)]).reshape(-1, CH)
    z2 = jnp.zeros((n_pad, d_h), jnp.float32)
    z1 = jnp.zeros((n_pad,), jnp.float32)

    seg_d = _seg_kernel(n_pad, e_pad, d_h, True)
    seg = _seg_kernel(n_pad, e_pad, d_h, False)
    seg1 = _seg1_kernel(n_pad, e_pad)

    f32 = jnp.float32
    k1 = pl.pallas_call(
        _k1_body,
        out_shape=(jax.ShapeDtypeStruct((n, d_h), f32),
                   jax.ShapeDtypeStruct((n, d_h), f32)))
    y1, r1 = k1(x, W1l.T, W1r.T, b1)

    part1, degp = seg_d(y1, src, dst, z2, z1)
    degp = degp.reshape(NC, n_pad)

    k2 = pl.pallas_call(
        functools.partial(_k2_body, n),
        out_shape=(jax.ShapeDtypeStruct((n, d_h), f32),
                   jax.ShapeDtypeStruct((n, d_h), f32),
                   jax.ShapeDtypeStruct((n, d_h), f32),
                   jax.ShapeDtypeStruct((n,), f32)))
    x1, y2, r2, inv = k2(part1, degp, r1, g1, be1, W2l.T, W2r.T, b2)

    part2 = seg(y2, src, dst, z2, z1)[0]

    k3 = pl.pallas_call(
        functools.partial(_k3_body, n),
        out_shape=(jax.ShapeDtypeStruct((n,), f32),
                   jax.ShapeDtypeStruct((n,), f32)))
    y3, r3 = k3(part2, inv, r2, x1, g2, be2, W3l, W3r, b3)

    part3 = seg1(y3, src, dst, z1).reshape(NC, n_pad)

    k4 = pl.pallas_call(
        functools.partial(_k4_body, n),
        out_shape=jax.ShapeDtypeStruct((n,), f32))
    return k4(part3, inv, r3)


# speedup vs baseline: 14.9522x; 1.8538x over previous
import functools

import jax
import jax.numpy as jnp
from jax import lax
from jax.experimental import pallas as pl
from jax.experimental.pallas import tpu as pltpu
from jax.experimental.pallas import tpu_sc as plsc

NC = 2
NS = 16
NW = NC * NS
CH = 128
NBUF = 2

_SC_PARAMS = pltpu.CompilerParams(use_tc_tiling_on_sc=False)


def _zchunks(zrows):
    out = []
    off = 0
    while off < zrows:
        out.append((off, min(CH, zrows - off)))
        off += CH
    return out


def _seg_kernel(n_pad, e_pad, d, with_deg):
    zrows = n_pad // NS
    wrows = e_pad // (NW * CH)
    ngroups = wrows // NBUF
    zck = _zchunks(zrows)
    mesh = plsc.VectorSubcoreMesh(core_axis_name="c", subcore_axis_name="s")

    out_type = [jax.ShapeDtypeStruct((NC, n_pad, d), jnp.float32)]
    scratch = [
        pltpu.VMEM_SHARED((n_pad, d), jnp.float32),
        pltpu.VMEM_SHARED((n_pad, d), jnp.float32),
        pltpu.VMEM((wrows, CH), jnp.int32),
        pltpu.VMEM((wrows, CH), jnp.int32),
    ]
    scratch += [pltpu.VMEM((CH, d), jnp.float32) for _ in range(NBUF)]
    scratch += [pltpu.SemaphoreType.DMA for _ in range(NBUF)]
    if with_deg:
        out_type.append(jax.ShapeDtypeStruct((NC * n_pad,), jnp.float32))
        scratch.append(pltpu.VMEM_SHARED((n_pad,), jnp.float32))
        scratch.append(pltpu.VMEM((CH,), jnp.float32))
        scratch.append(pltpu.VMEM((zrows,), jnp.float32))

    def body(y_h, src_h, dst_h, z2_h, part_h, *rest):
        if with_deg:
            deg_h = rest[0]
            rest = rest[1:]
        ycp, acc, si2, di2 = rest[:4]
        rows = rest[4:4 + NBUF]
        gsem = rest[4 + NBUF:4 + 2 * NBUF]
        if with_deg:
            dacc, ones, dstg = rest[4 + 2 * NBUF:]
        c = lax.axis_index("c")
        s = lax.axis_index("s")
        wid = s * NC + c
        r0 = s * zrows
        pltpu.sync_copy(z2_h.at[pl.ds(0, CH)], rows[1])
        for zo, zs in zck:
            pltpu.sync_copy(y_h.at[pl.ds(r0 + zo, zs)],
                            rows[0].at[pl.ds(0, zs)])
            pltpu.sync_copy(rows[0].at[pl.ds(0, zs)],
                            ycp.at[pl.ds(r0 + zo, zs)])
            pltpu.sync_copy(rows[1].at[pl.ds(0, zs)],
                            acc.at[pl.ds(r0 + zo, zs)])
        row0 = wid * wrows
        pltpu.sync_copy(src_h.at[pl.ds(row0, wrows)], si2)
        pltpu.sync_copy(dst_h.at[pl.ds(row0, wrows)], di2)
        if with_deg:
            for i in range(CH // 16):
                ones[pl.ds(i * 16, 16)] = jnp.ones((16,), jnp.float32)
            zd = jnp.zeros((16,), jnp.float32)
            for i in range(zrows // 16):
                dstg[pl.ds(i * 16, 16)] = zd
            dstg[pl.ds(zrows - 16, 16)] = zd
            pltpu.sync_copy(dstg, dacc.at[pl.ds(r0, zrows)])
        plsc.subcore_barrier()

        for b in range(NBUF):
            pltpu.async_copy(ycp.at[si2.at[b]], rows[b], gsem[b])

        def group(g, carry):
            for b in range(NBUF):
                j = g * NBUF + b
                pltpu.make_async_copy(ycp.at[si2.at[b]], rows[b],
                                      gsem[b]).wait()
                pltpu.sync_copy(rows[b], acc.at[di2.at[j]], add=True)
                if with_deg:
                    pltpu.sync_copy(ones, dacc.at[di2.at[j]], add=True)

                @pl.when(g < ngroups - 1)
                def _():
                    pltpu.async_copy(ycp.at[si2.at[j + NBUF]], rows[b],
                                     gsem[b])
            return carry

        lax.fori_loop(0, ngroups, group, 0)
        plsc.subcore_barrier()
        for zo, zs in zck:
            pltpu.sync_copy(acc.at[pl.ds(r0 + zo, zs)],
                            rows[0].at[pl.ds(0, zs)])
            pltpu.sync_copy(rows[0].at[pl.ds(0, zs)],
                            part_h.at[c, pl.ds(r0 + zo, zs)])
        if with_deg:
            pltpu.sync_copy(dacc.at[pl.ds(r0, zrows)], dstg)
            pltpu.sync_copy(dstg, deg_h.at[pl.ds(c * n_pad + r0, zrows)])

    return pl.kernel(body, out_type=tuple(out_type), mesh=mesh,
                     scratch_types=scratch, compiler_params=_SC_PARAMS)


def _seg1_kernel(n_pad, e_pad):
    zrows = n_pad // NS
    wrows = e_pad // (NW * CH)
    ngroups = wrows // NBUF
    mesh = plsc.VectorSubcoreMesh(core_axis_name="c", subcore_axis_name="s")

    scratch = [
        pltpu.VMEM_SHARED((n_pad,), jnp.float32),
        pltpu.VMEM_SHARED((n_pad,), jnp.float32),
        pltpu.VMEM((wrows, CH), jnp.int32),
        pltpu.VMEM((wrows, CH), jnp.int32),
        pltpu.VMEM((zrows,), jnp.float32),
    ]
    scratch += [pltpu.VMEM((CH,), jnp.float32) for _ in range(NBUF)]
    scratch += [pltpu.SemaphoreType.DMA for _ in range(NBUF)]

    def body(y_h, src_h, dst_h, part_h, ycp, acc, si2, di2, stg, *rest):
        vals = rest[:NBUF]
        gsem = rest[NBUF:]
        c = lax.axis_index("c")
        s = lax.axis_index("s")
        wid = s * NC + c
        r0 = s * zrows
        pltpu.sync_copy(y_h.at[pl.ds(r0, zrows)], stg)
        pltpu.sync_copy(stg, ycp.at[pl.ds(r0, zrows)])
        zd = jnp.zeros((16,), jnp.float32)
        for i in range(zrows // 16):
            stg[pl.ds(i * 16, 16)] = zd
        stg[pl.ds(zrows - 16, 16)] = zd
        pltpu.sync_copy(stg, acc.at[pl.ds(r0, zrows)])
        row0 = wid * wrows
        pltpu.sync_copy(src_h.at[pl.ds(row0, wrows)], si2)
        pltpu.sync_copy(dst_h.at[pl.ds(row0, wrows)], di2)
        plsc.subcore_barrier()

        for b in range(NBUF):
            pltpu.async_copy(ycp.at[si2.at[b]], vals[b], gsem[b])

        def group(g, carry):
            for b in range(NBUF):
                j = g * NBUF + b
                pltpu.make_async_copy(ycp.at[si2.at[b]], vals[b],
                                      gsem[b]).wait()
                pltpu.sync_copy(vals[b], acc.at[di2.at[j]], add=True)

                @pl.when(g < ngroups - 1)
                def _():
                    pltpu.async_copy(ycp.at[si2.at[j + NBUF]], vals[b],
                                     gsem[b])
            return carry

        lax.fori_loop(0, ngroups, group, 0)
        plsc.subcore_barrier()
        pltpu.sync_copy(acc.at[pl.ds(r0, zrows)], stg)
        pltpu.sync_copy(stg, part_h.at[pl.ds(c * n_pad + r0, zrows)])

    return pl.kernel(
        body,
        out_type=jax.ShapeDtypeStruct((NC * n_pad,), jnp.float32),
        mesh=mesh, scratch_types=scratch, compiler_params=_SC_PARAMS)


def _k1_body(n, n_pad, x_ref, wl_ref, wr_ref, b_ref, y_ref, r_ref):
    x = x_ref[...]
    y_ref[:n, :] = jnp.dot(x, wl_ref[...], preferred_element_type=jnp.float32)
    y_ref[n:, :] = jnp.zeros((n_pad - n, y_ref.shape[1]), jnp.float32)
    r_ref[...] = (jnp.dot(x, wr_ref[...], preferred_element_type=jnp.float32)
                  + b_ref[...][None, :])


def _k2_body(n, n_pad, p_ref, d_ref, r1_ref, g_ref, be_ref, wl_ref, wr_ref,
             b2_ref, x1_ref, y2_ref, r2_ref, inv_ref):
    agg = p_ref[0, :n, :] + p_ref[1, :n, :]
    deg = d_ref[0, :n] + d_ref[1, :n]
    inv = 1.0 / jnp.maximum(deg, 1.0)
    t = agg * inv[:, None] + r1_ref[...]
    mu = jnp.mean(t, axis=0)
    var = jnp.mean((t - mu[None, :]) ** 2, axis=0)
    xh = (t - mu[None, :]) * lax.rsqrt(var + 1e-5)[None, :]
    x1 = jnp.maximum(xh * g_ref[...][None, :] + be_ref[...][None, :], 0.0)
    x1_ref[...] = x1
    y2_ref[:n, :] = jnp.dot(x1, wl_ref[...],
                            preferred_element_type=jnp.float32)
    y2_ref[n:, :] = jnp.zeros((n_pad - n, y2_ref.shape[1]), jnp.float32)
    r2_ref[...] = (jnp.dot(x1, wr_ref[...], preferred_element_type=jnp.float32)
                   + b2_ref[...][None, :])
    inv_ref[...] = inv


def _k3_body(n, n_pad, p_ref, inv_ref, r2_ref, x1_ref, g_ref, be_ref, w3l_ref,
             w3r_ref, b3_ref, y3_ref, r3_ref):
    agg = p_ref[0, :n, :] + p_ref[1, :n, :]
    inv = inv_ref[...]
    t = agg * inv[:, None] + r2_ref[...]
    mu = jnp.mean(t, axis=0)
    var = jnp.mean((t - mu[None, :]) ** 2, axis=0)
    xh = (t - mu[None, :]) * lax.rsqrt(var + 1e-5)[None, :]
    x2 = jnp.maximum(xh * g_ref[...][None, :] + be_ref[...][None, :], 0.0)
    x2 = x2 + x1_ref[...]
    y3_ref[:n] = jnp.sum(x2 * w3l_ref[0][None, :], axis=1)
    y3_ref[n:] = jnp.zeros((n_pad - n,), jnp.float32)
    r3_ref[...] = jnp.sum(x2 * w3r_ref[0][None, :], axis=1) + b3_ref[0]


def _k4_body(n, p_ref, inv_ref, r3_ref, o_ref):
    agg = p_ref[0, :n] + p_ref[1, :n]
    o_ref[...] = agg * inv_ref[...] + r3_ref[...]


def kernel(x, edge_index, W1l, W1r, b1, g1, be1, W2l, W2r, b2, g2, be2,
           W3l, W3r, b3):
    n, d_in = x.shape
    d_h = W1l.shape[0]
    e = edge_index.shape[1]

    epw = -(-e // (NW * CH * NBUF)) * (CH * NBUF)
    e_pad = epw * NW
    n_pad = -(-(n + 1) // (NS * 8)) * (NS * 8)
    src = jnp.concatenate(
        [edge_index[0], jnp.zeros((e_pad - e,), jnp.int32)]).reshape(-1, CH)
    dst = jnp.concatenate(
        [edge_index[1], jnp.full((e_pad - e,), n, jnp.int32)]).reshape(-1, CH)
    z2 = jnp.zeros((n_pad, d_h), jnp.float32)

    seg_d = _seg_kernel(n_pad, e_pad, d_h, True)
    seg = _seg_kernel(n_pad, e_pad, d_h, False)
    seg1 = _seg1_kernel(n_pad, e_pad)

    f32 = jnp.float32
    k1 = pl.pallas_call(
        functools.partial(_k1_body, n, n_pad),
        out_shape=(jax.ShapeDtypeStruct((n_pad, d_h), f32),
                   jax.ShapeDtypeStruct((n, d_h), f32)))
    y1, r1 = k1(x, W1l.T, W1r.T, b1)

    part1, degp = seg_d(y1, src, dst, z2)
    degp = degp.reshape(NC, n_pad)

    k2 = pl.pallas_call(
        functools.partial(_k2_body, n, n_pad),
        out_shape=(jax.ShapeDtypeStruct((n, d_h), f32),
                   jax.ShapeDtypeStruct((n_pad, d_h), f32),
                   jax.ShapeDtypeStruct((n, d_h), f32),
                   jax.ShapeDtypeStruct((n,), f32)))
    x1, y2, r2, inv = k2(part1, degp, r1, g1, be1, W2l.T, W2r.T, b2)

    part2 = seg(y2, src, dst, z2)[0]

    k3 = pl.pallas_call(
        functools.partial(_k3_body, n, n_pad),
        out_shape=(jax.ShapeDtypeStruct((n_pad,), f32),
                   jax.ShapeDtypeStruct((n,), f32)))
    y3, r3 = k3(part2, inv, r2, x1, g2, be2, W3l, W3r, b3)

    part3 = seg1(y3, src, dst).reshape(NC, n_pad)

    k4 = pl.pallas_call(
        functools.partial(_k4_body, n),
        out_shape=jax.ShapeDtypeStruct((n,), f32))
    return k4(part3, inv, r3)


# speedup vs baseline: 15.1008x; 1.0099x over previous
import functools

import jax
import jax.numpy as jnp
from jax import lax
from jax.experimental import pallas as pl
from jax.experimental.pallas import tpu as pltpu
from jax.experimental.pallas import tpu_sc as plsc

NC = 2
NS = 16
NW = NC * NS
CH = 128
NBUF = 3

_SC_PARAMS = pltpu.CompilerParams(use_tc_tiling_on_sc=False)


def _zchunks(zrows):
    out = []
    off = 0
    while off < zrows:
        out.append((off, min(CH, zrows - off)))
        off += CH
    return out


def _seg_kernel(n_pad, e_pad, d, with_deg):
    zrows = n_pad // NS
    wrows = e_pad // (NW * CH)
    ngroups = wrows // NBUF
    zck = _zchunks(zrows)
    mesh = plsc.VectorSubcoreMesh(core_axis_name="c", subcore_axis_name="s")

    out_type = [jax.ShapeDtypeStruct((NC, n_pad, d), jnp.float32)]
    scratch = [
        pltpu.VMEM_SHARED((n_pad, d), jnp.float32),
        pltpu.VMEM_SHARED((n_pad, d), jnp.float32),
        pltpu.VMEM((wrows, CH), jnp.int32),
        pltpu.VMEM((wrows, CH), jnp.int32),
    ]
    scratch += [pltpu.VMEM((CH, d), jnp.float32) for _ in range(NBUF)]
    scratch += [pltpu.SemaphoreType.DMA for _ in range(NBUF)]
    if with_deg:
        out_type.append(jax.ShapeDtypeStruct((NC * n_pad,), jnp.float32))
        scratch.append(pltpu.VMEM_SHARED((n_pad,), jnp.float32))
        scratch.append(pltpu.VMEM((CH,), jnp.float32))
        scratch.append(pltpu.VMEM((zrows,), jnp.float32))
        scratch.append(pltpu.SemaphoreType.DMA)

    def body(y_h, src_h, dst_h, z2_h, part_h, *rest):
        if with_deg:
            deg_h = rest[0]
            rest = rest[1:]
        ycp, acc, si2, di2 = rest[:4]
        rows = rest[4:4 + NBUF]
        gsem = rest[4 + NBUF:4 + 2 * NBUF]
        if with_deg:
            dacc, ones, dstg, dsem = rest[4 + 2 * NBUF:]
        c = lax.axis_index("c")
        s = lax.axis_index("s")
        wid = s * NC + c
        r0 = s * zrows
        pltpu.sync_copy(z2_h.at[pl.ds(0, CH)], rows[1])
        for zo, zs in zck:
            pltpu.sync_copy(y_h.at[pl.ds(r0 + zo, zs)],
                            rows[0].at[pl.ds(0, zs)])
            pltpu.sync_copy(rows[0].at[pl.ds(0, zs)],
                            ycp.at[pl.ds(r0 + zo, zs)])
            pltpu.sync_copy(rows[1].at[pl.ds(0, zs)],
                            acc.at[pl.ds(r0 + zo, zs)])
        row0 = wid * wrows
        pltpu.sync_copy(src_h.at[pl.ds(row0, wrows)], si2)
        pltpu.sync_copy(dst_h.at[pl.ds(row0, wrows)], di2)
        if with_deg:
            for i in range(CH // 16):
                ones[pl.ds(i * 16, 16)] = jnp.ones((16,), jnp.float32)
            zd = jnp.zeros((16,), jnp.float32)
            for i in range(zrows // 16):
                dstg[pl.ds(i * 16, 16)] = zd
            dstg[pl.ds(zrows - 16, 16)] = zd
            pltpu.sync_copy(dstg, dacc.at[pl.ds(r0, zrows)])
        plsc.subcore_barrier()

        pltpu.async_copy(ycp.at[si2.at[0]], rows[0], gsem[0])
        pltpu.async_copy(ycp.at[si2.at[1]], rows[1], gsem[1])

        def deg_wait():
            pltpu.make_async_copy(ones, dacc.at[di2.at[0]], dsem).wait()

        def group(g, carry):
            for b in range(NBUF):
                j = g * NBUF + b
                q = (b + 2) % NBUF
                pltpu.make_async_copy(ycp.at[si2.at[b]], rows[b],
                                      gsem[b]).wait()
                if b == 0:
                    pltpu.async_copy(ycp.at[si2.at[j + 2]], rows[q], gsem[q])
                else:
                    @pl.when(g < ngroups - 1)
                    def _():
                        pltpu.async_copy(ycp.at[si2.at[j + 2]], rows[q],
                                         gsem[q])
                pltpu.sync_copy(rows[b], acc.at[di2.at[j]], add=True)
                if with_deg:
                    pltpu.async_copy(ones, dacc.at[di2.at[j]], dsem,
                                     add=True)

                    @pl.when(g >= 2)
                    def _():
                        deg_wait()
            return carry

        lax.fori_loop(0, ngroups, group, 0)
        if with_deg:
            for _ in range(2 * NBUF):
                deg_wait()
        plsc.subcore_barrier()
        for zo, zs in zck:
            pltpu.sync_copy(acc.at[pl.ds(r0 + zo, zs)],
                            rows[0].at[pl.ds(0, zs)])
            pltpu.sync_copy(rows[0].at[pl.ds(0, zs)],
                            part_h.at[c, pl.ds(r0 + zo, zs)])
        if with_deg:
            pltpu.sync_copy(dacc.at[pl.ds(r0, zrows)], dstg)
            pltpu.sync_copy(dstg, deg_h.at[pl.ds(c * n_pad + r0, zrows)])

    return pl.kernel(body, out_type=tuple(out_type), mesh=mesh,
                     scratch_types=scratch, compiler_params=_SC_PARAMS)


def _seg1_kernel(n_pad, e_pad):
    zrows = n_pad // NS
    wrows = e_pad // (NW * CH)
    ngroups = wrows // NBUF
    mesh = plsc.VectorSubcoreMesh(core_axis_name="c", subcore_axis_name="s")

    scratch = [
        pltpu.VMEM_SHARED((n_pad,), jnp.float32),
        pltpu.VMEM_SHARED((n_pad,), jnp.float32),
        pltpu.VMEM((wrows, CH), jnp.int32),
        pltpu.VMEM((wrows, CH), jnp.int32),
        pltpu.VMEM((zrows,), jnp.float32),
    ]
    scratch += [pltpu.VMEM((CH,), jnp.float32) for _ in range(NBUF)]
    scratch += [pltpu.SemaphoreType.DMA for _ in range(NBUF)]

    def body(y_h, src_h, dst_h, part_h, ycp, acc, si2, di2, stg, *rest):
        vals = rest[:NBUF]
        gsem = rest[NBUF:]
        c = lax.axis_index("c")
        s = lax.axis_index("s")
        wid = s * NC + c
        r0 = s * zrows
        pltpu.sync_copy(y_h.at[pl.ds(r0, zrows)], stg)
        pltpu.sync_copy(stg, ycp.at[pl.ds(r0, zrows)])
        zd = jnp.zeros((16,), jnp.float32)
        for i in range(zrows // 16):
            stg[pl.ds(i * 16, 16)] = zd
        stg[pl.ds(zrows - 16, 16)] = zd
        pltpu.sync_copy(stg, acc.at[pl.ds(r0, zrows)])
        row0 = wid * wrows
        pltpu.sync_copy(src_h.at[pl.ds(row0, wrows)], si2)
        pltpu.sync_copy(dst_h.at[pl.ds(row0, wrows)], di2)
        plsc.subcore_barrier()

        pltpu.async_copy(ycp.at[si2.at[0]], vals[0], gsem[0])
        pltpu.async_copy(ycp.at[si2.at[1]], vals[1], gsem[1])

        def group(g, carry):
            for b in range(NBUF):
                j = g * NBUF + b
                q = (b + 2) % NBUF
                pltpu.make_async_copy(ycp.at[si2.at[b]], vals[b],
                                      gsem[b]).wait()
                if b == 0:
                    pltpu.async_copy(ycp.at[si2.at[j + 2]], vals[q], gsem[q])
                else:
                    @pl.when(g < ngroups - 1)
                    def _():
                        pltpu.async_copy(ycp.at[si2.at[j + 2]], vals[q],
                                         gsem[q])
                pltpu.sync_copy(vals[b], acc.at[di2.at[j]], add=True)
            return carry

        lax.fori_loop(0, ngroups, group, 0)
        plsc.subcore_barrier()
        pltpu.sync_copy(acc.at[pl.ds(r0, zrows)], stg)
        pltpu.sync_copy(stg, part_h.at[pl.ds(c * n_pad + r0, zrows)])

    return pl.kernel(
        body,
        out_type=jax.ShapeDtypeStruct((NC * n_pad,), jnp.float32),
        mesh=mesh, scratch_types=scratch, compiler_params=_SC_PARAMS)


def _k1_body(n, n_pad, x_ref, wl_ref, wr_ref, b_ref, y_ref, r_ref):
    x = x_ref[...]
    y_ref[:n, :] = jnp.dot(x, wl_ref[...], preferred_element_type=jnp.float32)
    y_ref[n:, :] = jnp.zeros((n_pad - n, y_ref.shape[1]), jnp.float32)
    r_ref[...] = (jnp.dot(x, wr_ref[...], preferred_element_type=jnp.float32)
                  + b_ref[...][None, :])


def _k2_body(n, n_pad, p_ref, d_ref, r1_ref, g_ref, be_ref, wl_ref, wr_ref,
             b2_ref, x1_ref, y2_ref, r2_ref, inv_ref):
    agg = p_ref[0, :n, :] + p_ref[1, :n, :]
    deg = d_ref[0, :n] + d_ref[1, :n]
    inv = 1.0 / jnp.maximum(deg, 1.0)
    t = agg * inv[:, None] + r1_ref[...]
    mu = jnp.mean(t, axis=0)
    var = jnp.mean((t - mu[None, :]) ** 2, axis=0)
    xh = (t - mu[None, :]) * lax.rsqrt(var + 1e-5)[None, :]
    x1 = jnp.maximum(xh * g_ref[...][None, :] + be_ref[...][None, :], 0.0)
    x1_ref[...] = x1
    y2_ref[:n, :] = jnp.dot(x1, wl_ref[...],
                            preferred_element_type=jnp.float32)
    y2_ref[n:, :] = jnp.zeros((n_pad - n, y2_ref.shape[1]), jnp.float32)
    r2_ref[...] = (jnp.dot(x1, wr_ref[...], preferred_element_type=jnp.float32)
                   + b2_ref[...][None, :])
    inv_ref[...] = inv


def _k3_body(n, n_pad, p_ref, inv_ref, r2_ref, x1_ref, g_ref, be_ref, w3l_ref,
             w3r_ref, b3_ref, y3_ref, r3_ref):
    agg = p_ref[0, :n, :] + p_ref[1, :n, :]
    inv = inv_ref[...]
    t = agg * inv[:, None] + r2_ref[...]
    mu = jnp.mean(t, axis=0)
    var = jnp.mean((t - mu[None, :]) ** 2, axis=0)
    xh = (t - mu[None, :]) * lax.rsqrt(var + 1e-5)[None, :]
    x2 = jnp.maximum(xh * g_ref[...][None, :] + be_ref[...][None, :], 0.0)
    x2 = x2 + x1_ref[...]
    y3_ref[:n] = jnp.sum(x2 * w3l_ref[0][None, :], axis=1)
    y3_ref[n:] = jnp.zeros((n_pad - n,), jnp.float32)
    r3_ref[...] = jnp.sum(x2 * w3r_ref[0][None, :], axis=1) + b3_ref[0]


def _k4_body(n, p_ref, inv_ref, r3_ref, o_ref):
    agg = p_ref[0, :n] + p_ref[1, :n]
    o_ref[...] = agg * inv_ref[...] + r3_ref[...]


def kernel(x, edge_index, W1l, W1r, b1, g1, be1, W2l, W2r, b2, g2, be2,
           W3l, W3r, b3):
    n, d_in = x.shape
    d_h = W1l.shape[0]
    e = edge_index.shape[1]

    epw = -(-e // (NW * CH * NBUF)) * (CH * NBUF)
    e_pad = epw * NW
    n_pad = -(-(n + 1) // (NS * 8)) * (NS * 8)
    src = jnp.concatenate(
        [edge_index[0], jnp.zeros((e_pad - e,), jnp.int32)]).reshape(-1, CH)
    dst = jnp.concatenate(
        [edge_index[1], jnp.full((e_pad - e,), n, jnp.int32)]).reshape(-1, CH)
    z2 = jnp.zeros((n_pad, d_h), jnp.float32)

    seg_d = _seg_kernel(n_pad, e_pad, d_h, True)
    seg = _seg_kernel(n_pad, e_pad, d_h, False)
    seg1 = _seg1_kernel(n_pad, e_pad)

    f32 = jnp.float32
    k1 = pl.pallas_call(
        functools.partial(_k1_body, n, n_pad),
        out_shape=(jax.ShapeDtypeStruct((n_pad, d_h), f32),
                   jax.ShapeDtypeStruct((n, d_h), f32)))
    y1, r1 = k1(x, W1l.T, W1r.T, b1)

    part1, degp = seg_d(y1, src, dst, z2)
    degp = degp.reshape(NC, n_pad)

    k2 = pl.pallas_call(
        functools.partial(_k2_body, n, n_pad),
        out_shape=(jax.ShapeDtypeStruct((n, d_h), f32),
                   jax.ShapeDtypeStruct((n_pad, d_h), f32),
                   jax.ShapeDtypeStruct((n, d_h), f32),
                   jax.ShapeDtypeStruct((n,), f32)))
    x1, y2, r2, inv = k2(part1, degp, r1, g1, be1, W2l.T, W2r.T, b2)

    part2 = seg(y2, src, dst, z2)[0]

    k3 = pl.pallas_call(
        functools.partial(_k3_body, n, n_pad),
        out_shape=(jax.ShapeDtypeStruct((n_pad,), f32),
                   jax.ShapeDtypeStruct((n,), f32)))
    y3, r3 = k3(part2, inv, r2, x1, g2, be2, W3l, W3r, b3)

    part3 = seg1(y3, src, dst).reshape(NC, n_pad)

    k4 = pl.pallas_call(
        functools.partial(_k4_body, n),
        out_shape=jax.ShapeDtypeStruct((n,), f32))
    return k4(part3, inv, r3)


# speedup vs baseline: 16.1737x; 1.0710x over previous
import functools

import jax
import jax.numpy as jnp
from jax import lax
from jax.experimental import pallas as pl
from jax.experimental.pallas import tpu as pltpu
from jax.experimental.pallas import tpu_sc as plsc

NC = 2
NS = 16
NW = NC * NS
CH = 128
NBUF = 3

_SC_PARAMS = pltpu.CompilerParams(use_tc_tiling_on_sc=False)


def _zchunks(zrows):
    out = []
    off = 0
    while off < zrows:
        out.append((off, min(CH, zrows - off)))
        off += CH
    return out


def _fill(ref, nrows, d, vec16):
    def fi(i, carry):
        for jj in range(d // 16):
            ref[i, pl.ds(jj * 16, 16)] = vec16
        return carry
    lax.fori_loop(0, nrows, fi, 0)


def _seg_kernel(n_pad, nchunk, d, with_deg):
    zrows = n_pad // NS
    base = nchunk // NW
    left = nchunk - base * NW
    ngroups = base // NBUF
    assert base == ngroups * NBUF
    zck = _zchunks(zrows)
    mesh = plsc.VectorSubcoreMesh(core_axis_name="c", subcore_axis_name="s")

    out_type = [jax.ShapeDtypeStruct((NC, n_pad, d), jnp.float32)]
    scratch = [
        pltpu.VMEM_SHARED((n_pad, d), jnp.float32),
        pltpu.VMEM_SHARED((n_pad, d), jnp.float32),
        pltpu.VMEM((base + 1, CH), jnp.int32),
        pltpu.VMEM((base + 1, CH), jnp.int32),
    ]
    scratch += [pltpu.VMEM((CH, d), jnp.float32) for _ in range(NBUF)]
    scratch += [pltpu.SemaphoreType.DMA for _ in range(NBUF)]
    if with_deg:
        out_type.append(jax.ShapeDtypeStruct((NC * n_pad,), jnp.float32))
        scratch.append(pltpu.VMEM_SHARED((n_pad,), jnp.float32))
        scratch.append(pltpu.VMEM((CH,), jnp.float32))
        scratch.append(pltpu.VMEM((zrows,), jnp.float32))
        scratch.append(pltpu.SemaphoreType.DMA)

    def body(y_h, ei_h, part_h, *rest):
        if with_deg:
            deg_h = rest[0]
            rest = rest[1:]
        ycp, acc, si2, di2 = rest[:4]
        rows = rest[4:4 + NBUF]
        gsem = rest[4 + NBUF:4 + 2 * NBUF]
        if with_deg:
            dacc, ones, dstg, dsem = rest[4 + 2 * NBUF:]
        c = lax.axis_index("c")
        s = lax.axis_index("s")
        wid = s * NC + c
        r0 = s * zrows
        row0 = wid * base
        xrow = jnp.minimum(NW * base + wid, nchunk - 1)
        pltpu.sync_copy(ei_h.at[0, pl.ds(row0, base)],
                        si2.at[pl.ds(0, base)])
        pltpu.sync_copy(ei_h.at[1, pl.ds(row0, base)],
                        di2.at[pl.ds(0, base)])
        pltpu.sync_copy(ei_h.at[0, xrow], si2.at[base])
        pltpu.sync_copy(ei_h.at[1, xrow], di2.at[base])
        _fill(rows[1], CH, d, jnp.zeros((16,), jnp.float32))
        for zo, zs in zck:
            pltpu.sync_copy(y_h.at[pl.ds(r0 + zo, zs)],
                            rows[0].at[pl.ds(0, zs)])
            pltpu.sync_copy(rows[0].at[pl.ds(0, zs)],
                            ycp.at[pl.ds(r0 + zo, zs)])
            pltpu.sync_copy(rows[1].at[pl.ds(0, zs)],
                            acc.at[pl.ds(r0 + zo, zs)])
        if with_deg:
            for i in range(CH // 16):
                ones[pl.ds(i * 16, 16)] = jnp.ones((16,), jnp.float32)
            zd = jnp.zeros((16,), jnp.float32)
            for i in range(zrows // 16):
                dstg[pl.ds(i * 16, 16)] = zd
            dstg[pl.ds(zrows - 16, 16)] = zd
            pltpu.sync_copy(dstg, dacc.at[pl.ds(r0, zrows)])
        plsc.subcore_barrier()

        pltpu.async_copy(ycp.at[si2.at[0]], rows[0], gsem[0])
        pltpu.async_copy(ycp.at[si2.at[1]], rows[1], gsem[1])

        def deg_wait():
            pltpu.make_async_copy(ones, dacc.at[di2.at[0]], dsem).wait()

        def group(g, carry):
            for b in range(NBUF):
                j = g * NBUF + b
                q = (b + 2) % NBUF
                pltpu.make_async_copy(ycp.at[si2.at[b]], rows[b],
                                      gsem[b]).wait()
                if b == 0:
                    pltpu.async_copy(ycp.at[si2.at[j + 2]], rows[q], gsem[q])
                else:
                    @pl.when(g < ngroups - 1)
                    def _():
                        pltpu.async_copy(ycp.at[si2.at[j + 2]], rows[q],
                                         gsem[q])
                pltpu.sync_copy(rows[b], acc.at[di2.at[j]], add=True)
                if with_deg:
                    pltpu.async_copy(ones, dacc.at[di2.at[j]], dsem,
                                     add=True)

                    @pl.when(g >= 2)
                    def _():
                        deg_wait()
            return carry

        lax.fori_loop(0, ngroups, group, 0)
        if with_deg:
            for _ in range(2 * NBUF):
                deg_wait()

        @pl.when(wid < left)
        def _():
            pltpu.sync_copy(ycp.at[si2.at[base]], rows[0])
            pltpu.sync_copy(rows[0], acc.at[di2.at[base]], add=True)
            if with_deg:
                pltpu.sync_copy(ones, dacc.at[di2.at[base]], add=True)

        plsc.subcore_barrier()
        for zo, zs in zck:
            pltpu.sync_copy(acc.at[pl.ds(r0 + zo, zs)],
                            rows[0].at[pl.ds(0, zs)])
            pltpu.sync_copy(rows[0].at[pl.ds(0, zs)],
                            part_h.at[c, pl.ds(r0 + zo, zs)])
        if with_deg:
            pltpu.sync_copy(dacc.at[pl.ds(r0, zrows)], dstg)
            pltpu.sync_copy(dstg, deg_h.at[pl.ds(c * n_pad + r0, zrows)])

    return pl.kernel(body, out_type=tuple(out_type), mesh=mesh,
                     scratch_types=scratch, compiler_params=_SC_PARAMS)


def _seg1_kernel(n_pad, nchunk):
    zrows = n_pad // NS
    base = nchunk // NW
    left = nchunk - base * NW
    ngroups = base // NBUF
    mesh = plsc.VectorSubcoreMesh(core_axis_name="c", subcore_axis_name="s")

    scratch = [
        pltpu.VMEM_SHARED((n_pad,), jnp.float32),
        pltpu.VMEM_SHARED((n_pad,), jnp.float32),
        pltpu.VMEM((base + 1, CH), jnp.int32),
        pltpu.VMEM((base + 1, CH), jnp.int32),
        pltpu.VMEM((zrows,), jnp.float32),
    ]
    scratch += [pltpu.VMEM((CH,), jnp.float32) for _ in range(NBUF)]
    scratch += [pltpu.SemaphoreType.DMA for _ in range(NBUF)]

    def body(y_h, ei_h, part_h, ycp, acc, si2, di2, stg, *rest):
        vals = rest[:NBUF]
        gsem = rest[NBUF:]
        c = lax.axis_index("c")
        s = lax.axis_index("s")
        wid = s * NC + c
        r0 = s * zrows
        row0 = wid * base
        xrow = jnp.minimum(NW * base + wid, nchunk - 1)
        pltpu.sync_copy(ei_h.at[0, pl.ds(row0, base)],
                        si2.at[pl.ds(0, base)])
        pltpu.sync_copy(ei_h.at[1, pl.ds(row0, base)],
                        di2.at[pl.ds(0, base)])
        pltpu.sync_copy(ei_h.at[0, xrow], si2.at[base])
        pltpu.sync_copy(ei_h.at[1, xrow], di2.at[base])
        pltpu.sync_copy(y_h.at[pl.ds(r0, zrows)], stg)
        pltpu.sync_copy(stg, ycp.at[pl.ds(r0, zrows)])
        zd = jnp.zeros((16,), jnp.float32)
        for i in range(zrows // 16):
            stg[pl.ds(i * 16, 16)] = zd
        stg[pl.ds(zrows - 16, 16)] = zd
        pltpu.sync_copy(stg, acc.at[pl.ds(r0, zrows)])
        plsc.subcore_barrier()

        pltpu.async_copy(ycp.at[si2.at[0]], vals[0], gsem[0])
        pltpu.async_copy(ycp.at[si2.at[1]], vals[1], gsem[1])

        def group(g, carry):
            for b in range(NBUF):
                j = g * NBUF + b
                q = (b + 2) % NBUF
                pltpu.make_async_copy(ycp.at[si2.at[b]], vals[b],
                                      gsem[b]).wait()
                if b == 0:
                    pltpu.async_copy(ycp.at[si2.at[j + 2]], vals[q], gsem[q])
                else:
                    @pl.when(g < ngroups - 1)
                    def _():
                        pltpu.async_copy(ycp.at[si2.at[j + 2]], vals[q],
                                         gsem[q])
                pltpu.sync_copy(vals[b], acc.at[di2.at[j]], add=True)
            return carry

        lax.fori_loop(0, ngroups, group, 0)

        @pl.when(wid < left)
        def _():
            pltpu.sync_copy(ycp.at[si2.at[base]], vals[0])
            pltpu.sync_copy(vals[0], acc.at[di2.at[base]], add=True)

        plsc.subcore_barrier()
        pltpu.sync_copy(acc.at[pl.ds(r0, zrows)], stg)
        pltpu.sync_copy(stg, part_h.at[pl.ds(c * n_pad + r0, zrows)])

    return pl.kernel(
        body,
        out_type=jax.ShapeDtypeStruct((NC * n_pad,), jnp.float32),
        mesh=mesh, scratch_types=scratch, compiler_params=_SC_PARAMS)


def _k1_body(n, n_pad, x_ref, wl_ref, wr_ref, b_ref, y_ref, r_ref):
    x = x_ref[...]
    y_ref[:n, :] = jnp.dot(x, wl_ref[...], preferred_element_type=jnp.float32)
    y_ref[n:, :] = jnp.zeros((n_pad - n, y_ref.shape[1]), jnp.float32)
    r_ref[...] = (jnp.dot(x, wr_ref[...], preferred_element_type=jnp.float32)
                  + b_ref[...][None, :])


def _k2_body(n, n_pad, p_ref, d_ref, r1_ref, g_ref, be_ref, wl_ref,
             x1_ref, y2_ref, inv_ref):
    agg = p_ref[0, :n, :] + p_ref[1, :n, :]
    deg = d_ref[0, :n] + d_ref[1, :n]
    inv = 1.0 / jnp.maximum(deg, 1.0)
    t = agg * inv[:, None] + r1_ref[...]
    mu = jnp.mean(t, axis=0)
    var = jnp.mean((t - mu[None, :]) ** 2, axis=0)
    xh = (t - mu[None, :]) * lax.rsqrt(var + 1e-5)[None, :]
    x1 = jnp.maximum(xh * g_ref[...][None, :] + be_ref[...][None, :], 0.0)
    x1_ref[...] = x1
    y2_ref[:n, :] = jnp.dot(x1, wl_ref[...],
                            preferred_element_type=jnp.float32)
    y2_ref[n:, :] = jnp.zeros((n_pad - n, y2_ref.shape[1]), jnp.float32)
    inv_ref[...] = inv


def _k3_body(n, n_pad, p_ref, inv_ref, x1_ref, wr_ref, b2_ref, g_ref, be_ref,
             w3l_ref, w3r_ref, b3_ref, y3_ref, r3_ref):
    agg = p_ref[0, :n, :] + p_ref[1, :n, :]
    inv = inv_ref[...]
    x1 = x1_ref[...]
    r2 = (jnp.dot(x1, wr_ref[...], preferred_element_type=jnp.float32)
          + b2_ref[...][None, :])
    t = agg * inv[:, None] + r2
    mu = jnp.mean(t, axis=0)
    var = jnp.mean((t - mu[None, :]) ** 2, axis=0)
    xh = (t - mu[None, :]) * lax.rsqrt(var + 1e-5)[None, :]
    x2 = jnp.maximum(xh * g_ref[...][None, :] + be_ref[...][None, :], 0.0)
    x2 = x2 + x1
    y3_ref[:n] = jnp.sum(x2 * w3l_ref[0][None, :], axis=1)
    y3_ref[n:] = jnp.zeros((n_pad - n,), jnp.float32)
    r3_ref[...] = jnp.sum(x2 * w3r_ref[0][None, :], axis=1) + b3_ref[0]


def _k4_body(n, p_ref, inv_ref, r3_ref, o_ref):
    agg = p_ref[0, :n] + p_ref[1, :n]
    o_ref[...] = agg * inv_ref[...] + r3_ref[...]


def kernel(x, edge_index, W1l, W1r, b1, g1, be1, W2l, W2r, b2, g2, be2,
           W3l, W3r, b3):
    n, d_in = x.shape
    d_h = W1l.shape[0]
    e = edge_index.shape[1]
    assert e % CH == 0
    nchunk = e // CH
    n_pad = -(-(n + 1) // (NS * 8)) * (NS * 8)
    ei = edge_index.reshape(2, nchunk, CH)

    seg_d = _seg_kernel(n_pad, nchunk, d_h, True)
    seg = _seg_kernel(n_pad, nchunk, d_h, False)
    seg1 = _seg1_kernel(n_pad, nchunk)

    f32 = jnp.float32
    k1 = pl.pallas_call(
        functools.partial(_k1_body, n, n_pad),
        out_shape=(jax.ShapeDtypeStruct((n_pad, d_h), f32),
                   jax.ShapeDtypeStruct((n, d_h), f32)))
    y1, r1 = k1(x, W1l.T, W1r.T, b1)

    part1, degp = seg_d(y1, ei)
    degp = degp.reshape(NC, n_pad)

    k2 = pl.pallas_call(
        functools.partial(_k2_body, n, n_pad),
        out_shape=(jax.ShapeDtypeStruct((n, d_h), f32),
                   jax.ShapeDtypeStruct((n_pad, d_h), f32),
                   jax.ShapeDtypeStruct((n,), f32)))
    x1, y2, inv = k2(part1, degp, r1, g1, be1, W2l.T)

    part2 = seg(y2, ei)[0]

    k3 = pl.pallas_call(
        functools.partial(_k3_body, n, n_pad),
        out_shape=(jax.ShapeDtypeStruct((n_pad,), f32),
                   jax.ShapeDtypeStruct((n,), f32)))
    y3, r3 = k3(part2, inv, x1, W2r.T, b2, g2, be2, W3l, W3r, b3)

    part3 = seg1(y3, ei).reshape(NC, n_pad)

    k4 = pl.pallas_call(
        functools.partial(_k4_body, n),
        out_shape=jax.ShapeDtypeStruct((n,), f32))
    return k4(part3, inv, r3)
